# Initial kernel scaffold; baseline (speedup 1.0000x reference)
import functools

import jax
import jax.numpy as jnp
from jax import lax
from jax.experimental import pallas as pl
from jax.experimental.pallas import tpu as pltpu
from jax.experimental.pallas import tpu_sc as plsc

B = 16384
D = 128
NC, NS, L = 2, 16, 16
NW = NC * NS
BPW = B // NW
CH = 128

SID_ROWS = 1016
SID_OFF = 144
EID_OFF = 144 + SID_ROWS
T_ROWS = 144 + 2 * SID_ROWS


def _proj_body(dep_ref, sid_ref, eid_ref, w_ref, t_ref):
    t_ref[0:144, :] = jnp.dot(dep_ref[...], w_ref[0:256, :],
                              preferred_element_type=jnp.float32)
    t_ref[144:144 + SID_ROWS, :] = jnp.dot(sid_ref[...], w_ref[256:512, :],
                                           preferred_element_type=jnp.float32)
    t_ref[EID_OFF:T_ROWS, :] = jnp.dot(eid_ref[...], w_ref[512:768, :],
                                       preferred_element_type=jnp.float32)


_mesh = plsc.VectorSubcoreMesh(core_axis_name="c", subcore_axis_name="s")


@functools.partial(
    pl.kernel,
    out_type=jax.ShapeDtypeStruct((B, D), jnp.float32),
    mesh=_mesh,
    scratch_types=[
        pltpu.VMEM((BPW * 5,), jnp.float32),
        pltpu.VMEM((BPW,), jnp.int32),
        pltpu.VMEM((BPW,), jnp.int32),
        pltpu.VMEM((BPW,), jnp.int32),
        pltpu.VMEM((CH, D), jnp.float32),
        pltpu.VMEM((CH, D), jnp.float32),
        pltpu.VMEM((CH, D), jnp.float32),
        pltpu.SemaphoreType.DMA,
        pltpu.SemaphoreType.DMA,
        pltpu.SemaphoreType.DMA,
    ],
)
def _gather_sum(t_hbm, attr_hbm, g_hbm,
                attr_v, idx_d, idx_s, idx_e, b1, b2, b3, s1, s2, s3):
    wid = lax.axis_index("s") * NC + lax.axis_index("c")
    base = wid * BPW
    pltpu.sync_copy(attr_hbm.at[pl.ds(base * 5, BPW * 5)], attr_v)

    def build(i, carry):
        flat = (lax.iota(jnp.int32, (L,)) + i * L) * 5
        d = plsc.load_gather(attr_v, [flat]).astype(jnp.int32)
        s = plsc.load_gather(attr_v, [flat + 3]).astype(jnp.int32) + SID_OFF
        e = plsc.load_gather(attr_v, [flat + 4]).astype(jnp.int32) + EID_OFF
        idx_d[pl.ds(i * L, L)] = d
        idx_s[pl.ds(i * L, L)] = s
        idx_e[pl.ds(i * L, L)] = e
        return carry

    lax.fori_loop(0, BPW // L, build, 0)

    for t in range(BPW // CH):
        cb = t * CH
        c1 = pltpu.async_copy(t_hbm.at[idx_d.at[pl.ds(cb, CH)]], b1, s1)
        c2 = pltpu.async_copy(t_hbm.at[idx_s.at[pl.ds(cb, CH)]], b2, s2)
        c3 = pltpu.async_copy(t_hbm.at[idx_e.at[pl.ds(cb, CH)]], b3, s3)
        c1.wait()
        c2.wait()
        c3.wait()

        def sum_body(r, carry):
            for v in range(D // L):
                sl = pl.ds(v * L, L)
                b1[r, sl] = b1[r, sl] + b2[r, sl] + b3[r, sl]
            return carry

        lax.fori_loop(0, CH, sum_body, 0)
        pltpu.sync_copy(b1, g_hbm.at[pl.ds(base + cb, CH), :])


def _final_body(g_ref, cont_ref, fc1b_ref, fc2w_ref, ww_ref, fc2b_ref,
                wb_ref, o_ref):
    h = jnp.maximum(g_ref[...] + fc1b_ref[...], 0.0)
    o_ref[...] = (jnp.dot(h, fc2w_ref[...], preferred_element_type=jnp.float32)
                  + jnp.dot(cont_ref[...], ww_ref[...],
                            preferred_element_type=jnp.float32)
                  + fc2b_ref[...] + wb_ref[...])


BLK = 2048


def kernel(attr, wide_W, wide_b, dep_table, sid_table, eid_table,
           fc1_W, fc1_b, fc2_W, fc2_b):
    sid_p = jnp.pad(sid_table, ((0, SID_ROWS - sid_table.shape[0]), (0, 0)))
    eid_p = jnp.pad(eid_table, ((0, SID_ROWS - eid_table.shape[0]), (0, 0)))

    t_proj = pl.pallas_call(
        _proj_body,
        out_shape=jax.ShapeDtypeStruct((T_ROWS, D), jnp.float32),
    )(dep_table, sid_p, eid_p, fc1_W)

    g = _gather_sum(t_proj, attr.reshape(-1))

    cont = attr[:, 1:3]
    out = pl.pallas_call(
        _final_body,
        grid=(B // BLK,),
        in_specs=[
            pl.BlockSpec((BLK, D), lambda i: (i, 0)),
            pl.BlockSpec((BLK, 2), lambda i: (i, 0)),
            pl.BlockSpec((1, D), lambda i: (0, 0)),
            pl.BlockSpec((D, D), lambda i: (0, 0)),
            pl.BlockSpec((2, D), lambda i: (0, 0)),
            pl.BlockSpec((1, D), lambda i: (0, 0)),
            pl.BlockSpec((1, D), lambda i: (0, 0)),
        ],
        out_specs=pl.BlockSpec((BLK, D), lambda i: (i, 0)),
        out_shape=jax.ShapeDtypeStruct((B, D), jnp.float32),
    )(g, cont, fc1_b.reshape(1, D), fc2_W, wide_W, fc2_b.reshape(1, D),
      wide_b.reshape(1, D))
    return out


# speedup vs baseline: 3.0480x; 3.0480x over previous
import functools

import jax
import jax.numpy as jnp
from jax import lax
from jax.experimental import pallas as pl
from jax.experimental.pallas import tpu as pltpu
from jax.experimental.pallas import tpu_sc as plsc

B = 16384
D = 128
NC, NS, L = 2, 16, 16
NW = NC * NS
BPW = B // NW
CH = 128

SID_ROWS = 1016
SID_OFF = 144
EID_OFF = 144 + SID_ROWS
T_ROWS = 144 + 2 * SID_ROWS


def _proj_body(dep_ref, sid_ref, eid_ref, w_ref, t_ref):
    t_ref[0:144, :] = jnp.dot(dep_ref[...], w_ref[0:256, :],
                              preferred_element_type=jnp.float32)
    t_ref[144:144 + SID_ROWS, :] = jnp.dot(sid_ref[...], w_ref[256:512, :],
                                           preferred_element_type=jnp.float32)
    t_ref[EID_OFF:T_ROWS, :] = jnp.dot(eid_ref[...], w_ref[512:768, :],
                                       preferred_element_type=jnp.float32)


_mesh = plsc.VectorSubcoreMesh(core_axis_name="c", subcore_axis_name="s",
                               num_cores=NC, num_subcores=NS)


@functools.partial(
    pl.kernel,
    out_type=jax.ShapeDtypeStruct((B, D), jnp.float32),
    mesh=_mesh,
    scratch_types=[
        pltpu.VMEM((BPW,), jnp.float32),
        pltpu.VMEM((BPW,), jnp.float32),
        pltpu.VMEM((BPW,), jnp.float32),
        pltpu.VMEM((BPW,), jnp.int32),
        pltpu.VMEM((BPW,), jnp.int32),
        pltpu.VMEM((BPW,), jnp.int32),
        pltpu.VMEM((CH, D), jnp.float32),
        pltpu.VMEM((CH, D), jnp.float32),
        pltpu.VMEM((CH, D), jnp.float32),
        pltpu.SemaphoreType.DMA,
        pltpu.SemaphoreType.DMA,
        pltpu.SemaphoreType.DMA,
    ],
)
def _gather_sum(t_hbm, ad_hbm, as_hbm, ae_hbm, g_hbm,
                col_d, col_s, col_e, idx_d, idx_s, idx_e,
                b1, b2, b3, s1, s2, s3):
    wid = lax.axis_index("s") * NC + lax.axis_index("c")
    base = wid * BPW
    pltpu.sync_copy(ad_hbm.at[pl.ds(base, BPW)], col_d)
    pltpu.sync_copy(as_hbm.at[pl.ds(base, BPW)], col_s)
    pltpu.sync_copy(ae_hbm.at[pl.ds(base, BPW)], col_e)

    def build(i, carry):
        sl = pl.ds(i * L, L)
        idx_d[sl] = col_d[sl].astype(jnp.int32)
        idx_s[sl] = col_s[sl].astype(jnp.int32) + SID_OFF
        idx_e[sl] = col_e[sl].astype(jnp.int32) + EID_OFF
        return carry

    lax.fori_loop(0, BPW // L, build, 0)

    for t in range(BPW // CH):
        cb = t * CH
        c1 = pltpu.async_copy(t_hbm.at[idx_d.at[pl.ds(cb, CH)]], b1, s1)
        c2 = pltpu.async_copy(t_hbm.at[idx_s.at[pl.ds(cb, CH)]], b2, s2)
        c3 = pltpu.async_copy(t_hbm.at[idx_e.at[pl.ds(cb, CH)]], b3, s3)
        c1.wait()
        c2.wait()
        c3.wait()

        def sum_body(r, carry):
            for v in range(D // L):
                sl = pl.ds(v * L, L)
                b1[r, sl] = b1[r, sl] + b2[r, sl] + b3[r, sl]
            return carry

        lax.fori_loop(0, CH, sum_body, 0)
        pltpu.sync_copy(b1, g_hbm.at[pl.ds(base + cb, CH), :])


def _final_body(g_ref, cont_ref, fc1b_ref, fc2w_ref, ww_ref, fc2b_ref,
                wb_ref, o_ref):
    h = jnp.maximum(g_ref[...] + fc1b_ref[...], 0.0)
    o_ref[...] = (jnp.dot(h, fc2w_ref[...], preferred_element_type=jnp.float32)
                  + jnp.dot(cont_ref[...], ww_ref[...],
                            preferred_element_type=jnp.float32)
                  + fc2b_ref[...] + wb_ref[...])


BLK = 2048


def kernel(attr, wide_W, wide_b, dep_table, sid_table, eid_table,
           fc1_W, fc1_b, fc2_W, fc2_b):
    sid_p = jnp.pad(sid_table, ((0, SID_ROWS - sid_table.shape[0]), (0, 0)))
    eid_p = jnp.pad(eid_table, ((0, SID_ROWS - eid_table.shape[0]), (0, 0)))

    t_proj = pl.pallas_call(
        _proj_body,
        out_shape=jax.ShapeDtypeStruct((T_ROWS, D), jnp.float32),
    )(dep_table, sid_p, eid_p, fc1_W)

    g = _gather_sum(t_proj, attr[:, 0], attr[:, 3], attr[:, 4])

    cont = attr[:, 1:3]
    out = pl.pallas_call(
        _final_body,
        grid=(B // BLK,),
        in_specs=[
            pl.BlockSpec((BLK, D), lambda i: (i, 0)),
            pl.BlockSpec((BLK, 2), lambda i: (i, 0)),
            pl.BlockSpec((1, D), lambda i: (0, 0)),
            pl.BlockSpec((D, D), lambda i: (0, 0)),
            pl.BlockSpec((2, D), lambda i: (0, 0)),
            pl.BlockSpec((1, D), lambda i: (0, 0)),
            pl.BlockSpec((1, D), lambda i: (0, 0)),
        ],
        out_specs=pl.BlockSpec((BLK, D), lambda i: (i, 0)),
        out_shape=jax.ShapeDtypeStruct((B, D), jnp.float32),
    )(g, cont, fc1_b.reshape(1, D), fc2_W, wide_W, fc2_b.reshape(1, D),
      wide_b.reshape(1, D))
    return out


# speedup vs baseline: 3.3336x; 1.0937x over previous
import functools

import jax
import jax.numpy as jnp
from jax import lax
from jax.experimental import pallas as pl
from jax.experimental.pallas import tpu as pltpu
from jax.experimental.pallas import tpu_sc as plsc

B = 16384
D = 128
NC, NS, L = 2, 16, 16
NW = NC * NS
BPW = B // NW
CH = 128
NCHUNK = BPW // CH


def _proj_body(dep_ref, sid_ref, eid_ref, w_ref, t1_ref, t2_ref, t3_ref):
    t1_ref[...] = jnp.dot(dep_ref[...], w_ref[0:256, :],
                          preferred_element_type=jnp.float32)
    t2_ref[...] = jnp.dot(sid_ref[...], w_ref[256:512, :],
                          preferred_element_type=jnp.float32)
    t3_ref[...] = jnp.dot(eid_ref[...], w_ref[512:768, :],
                          preferred_element_type=jnp.float32)


_mesh = plsc.VectorSubcoreMesh(core_axis_name="c", subcore_axis_name="s",
                               num_cores=NC, num_subcores=NS)


@functools.partial(
    pl.kernel,
    out_type=jax.ShapeDtypeStruct((B, D), jnp.float32),
    mesh=_mesh,
    scratch_types=[
        pltpu.VMEM((BPW,), jnp.float32),
        pltpu.VMEM((BPW,), jnp.float32),
        pltpu.VMEM((BPW,), jnp.float32),
        pltpu.VMEM((BPW,), jnp.int32),
        pltpu.VMEM((BPW,), jnp.int32),
        pltpu.VMEM((BPW,), jnp.int32),
        pltpu.VMEM((CH, D), jnp.float32),
        pltpu.VMEM((CH, D), jnp.float32),
        pltpu.VMEM((CH, D), jnp.float32),
        pltpu.VMEM((CH, D), jnp.float32),
        pltpu.VMEM((CH, D), jnp.float32),
        pltpu.VMEM((CH, D), jnp.float32),
        pltpu.SemaphoreType.DMA,
        pltpu.SemaphoreType.DMA,
        pltpu.SemaphoreType.DMA,
        pltpu.SemaphoreType.DMA,
    ],
)
def _gather_sum(t1_hbm, t2_hbm, t3_hbm, at_hbm, g_hbm,
                col_d, col_s, col_e, idx_d, idx_s, idx_e,
                a1, a2, a3, b1, b2, b3, gs0, gs1, ws0, ws1):
    wid = lax.axis_index("s") * NC + lax.axis_index("c")
    base = wid * BPW
    pltpu.sync_copy(at_hbm.at[pl.ds(0 * B + base, BPW)], col_d)
    pltpu.sync_copy(at_hbm.at[pl.ds(3 * B + base, BPW)], col_s)
    pltpu.sync_copy(at_hbm.at[pl.ds(4 * B + base, BPW)], col_e)

    def build(i, carry):
        sl = pl.ds(i * L, L)
        idx_d[sl] = col_d[sl].astype(jnp.int32)
        idx_s[sl] = col_s[sl].astype(jnp.int32)
        idx_e[sl] = col_e[sl].astype(jnp.int32)
        return carry

    lax.fori_loop(0, BPW // L, build, 0)

    bufs = ((a1, a2, a3), (b1, b2, b3))
    gsems = (gs0, gs1)
    wsems = (ws0, ws1)

    def fire(t, p):
        cb = t * CH
        s = gsems[p]
        u1, u2, u3 = bufs[p]
        return (pltpu.async_copy(t1_hbm.at[idx_d.at[pl.ds(cb, CH)]], u1, s),
                pltpu.async_copy(t2_hbm.at[idx_s.at[pl.ds(cb, CH)]], u2, s),
                pltpu.async_copy(t3_hbm.at[idx_e.at[pl.ds(cb, CH)]], u3, s))

    pend = {0: fire(0, 0)}
    wb = {}
    for t in range(NCHUNK):
        p = t % 2
        if t + 1 < NCHUNK:
            pn = (t + 1) % 2
            if pn in wb:
                wb.pop(pn).wait()
            pend[pn] = fire(t + 1, pn)
        for c in pend.pop(p):
            c.wait()
        u1, u2, u3 = bufs[p]

        def sum_body(r, carry):
            for v in range(D // L):
                sl = pl.ds(v * L, L)
                u1[r, sl] = u1[r, sl] + u2[r, sl] + u3[r, sl]
            return carry

        lax.fori_loop(0, CH, sum_body, 0)
        wb[p] = pltpu.async_copy(u1, g_hbm.at[pl.ds(base + t * CH, CH), :],
                                 wsems[p])
    for c in wb.values():
        c.wait()


def _final_body(g_ref, cont_ref, fc1b_ref, fc2w_ref, ww_ref, fc2b_ref,
                wb_ref, o_ref):
    h = jnp.maximum(g_ref[...] + fc1b_ref[...], 0.0)
    o_ref[...] = (jnp.dot(h, fc2w_ref[...], preferred_element_type=jnp.float32)
                  + jnp.dot(cont_ref[...], ww_ref[...],
                            preferred_element_type=jnp.float32)
                  + fc2b_ref[...] + wb_ref[...])


BLK = 2048


def kernel(attr, wide_W, wide_b, dep_table, sid_table, eid_table,
           fc1_W, fc1_b, fc2_W, fc2_b):
    n_sid = sid_table.shape[0]
    t1, t2, t3 = pl.pallas_call(
        _proj_body,
        out_shape=[jax.ShapeDtypeStruct((dep_table.shape[0], D), jnp.float32),
                   jax.ShapeDtypeStruct((n_sid, D), jnp.float32),
                   jax.ShapeDtypeStruct((n_sid, D), jnp.float32)],
    )(dep_table, sid_table, eid_table, fc1_W)

    attr_t = attr.T.reshape(-1)
    g = _gather_sum(t1, t2, t3, attr_t)

    cont = attr[:, 1:3]
    out = pl.pallas_call(
        _final_body,
        grid=(B // BLK,),
        in_specs=[
            pl.BlockSpec((BLK, D), lambda i: (i, 0)),
            pl.BlockSpec((BLK, 2), lambda i: (i, 0)),
            pl.BlockSpec((1, D), lambda i: (0, 0)),
            pl.BlockSpec((D, D), lambda i: (0, 0)),
            pl.BlockSpec((2, D), lambda i: (0, 0)),
            pl.BlockSpec((1, D), lambda i: (0, 0)),
            pl.BlockSpec((1, D), lambda i: (0, 0)),
        ],
        out_specs=pl.BlockSpec((BLK, D), lambda i: (i, 0)),
        out_shape=jax.ShapeDtypeStruct((B, D), jnp.float32),
    )(g, cont, fc1_b.reshape(1, D), fc2_W, wide_W, fc2_b.reshape(1, D),
      wide_b.reshape(1, D))
    return out


# speedup vs baseline: 4.5703x; 1.3710x over previous
import functools

import jax
import jax.numpy as jnp
from jax import lax
from jax.experimental import pallas as pl
from jax.experimental.pallas import tpu as pltpu
from jax.experimental.pallas import tpu_sc as plsc

B = 16384
D = 128
N_DEP = 144
N_SID = 1015
NC, NS, L = 2, 16, 16
NW = NC * NS
BPW = B // NW
CH = 128
NCHUNK = BPW // CH


def _proj_body(dep_ref, sid_ref, eid_ref, w_ref, t1_ref, t2_ref, t3_ref):
    t1_ref[...] = jnp.dot(dep_ref[...], w_ref[0:256, :],
                          preferred_element_type=jnp.float32)
    t2_ref[...] = jnp.dot(sid_ref[...], w_ref[256:512, :],
                          preferred_element_type=jnp.float32)
    t3_ref[...] = jnp.dot(eid_ref[...], w_ref[512:768, :],
                          preferred_element_type=jnp.float32)


_mesh = plsc.VectorSubcoreMesh(core_axis_name="c", subcore_axis_name="s",
                               num_cores=NC, num_subcores=NS)


@functools.partial(
    pl.kernel,
    out_type=jax.ShapeDtypeStruct((B, D), jnp.float32),
    mesh=_mesh,
    scratch_types=[
        pltpu.VMEM_SHARED((N_DEP, D), jnp.float32),
        pltpu.VMEM_SHARED((N_SID, D), jnp.float32),
        pltpu.VMEM_SHARED((N_SID, D), jnp.float32),
        pltpu.VMEM((BPW,), jnp.float32),
        pltpu.VMEM((BPW,), jnp.float32),
        pltpu.VMEM((BPW,), jnp.float32),
        pltpu.VMEM((BPW,), jnp.int32),
        pltpu.VMEM((BPW,), jnp.int32),
        pltpu.VMEM((BPW,), jnp.int32),
        pltpu.VMEM((CH, D), jnp.float32),
        pltpu.VMEM((CH, D), jnp.float32),
        pltpu.VMEM((CH, D), jnp.float32),
        pltpu.VMEM((CH, D), jnp.float32),
        pltpu.VMEM((CH, D), jnp.float32),
        pltpu.VMEM((CH, D), jnp.float32),
        pltpu.SemaphoreType.DMA,
        pltpu.SemaphoreType.DMA,
        pltpu.SemaphoreType.DMA,
        pltpu.SemaphoreType.DMA,
        pltpu.SemaphoreType.DMA,
    ],
)
def _gather_sum(t1_hbm, t2_hbm, t3_hbm, at_hbm, g_hbm,
                ts1, ts2, ts3,
                col_d, col_s, col_e, idx_d, idx_s, idx_e,
                a1, a2, a3, b1, b2, b3, sst, gs0, gs1, ws0, ws1):
    cid = lax.axis_index("c")
    sid = lax.axis_index("s")
    wid = sid * NC + cid
    base = wid * BPW

    @pl.when(sid == 0)
    def _stage():
        pltpu.async_copy(t1_hbm, ts1, sst)
        pltpu.async_copy(t2_hbm, ts2, sst)
        c3 = pltpu.async_copy(t3_hbm, ts3, sst)
        del c3

    pltpu.sync_copy(at_hbm.at[pl.ds(0 * B + base, BPW)], col_d)
    pltpu.sync_copy(at_hbm.at[pl.ds(3 * B + base, BPW)], col_s)
    pltpu.sync_copy(at_hbm.at[pl.ds(4 * B + base, BPW)], col_e)

    def build(i, carry):
        sl = pl.ds(i * L, L)
        idx_d[sl] = col_d[sl].astype(jnp.int32)
        idx_s[sl] = col_s[sl].astype(jnp.int32)
        idx_e[sl] = col_e[sl].astype(jnp.int32)
        return carry

    lax.fori_loop(0, BPW // L, build, 0)

    @pl.when(sid == 0)
    def _stage_wait():
        pltpu.make_async_copy(t1_hbm, ts1, sst).wait()
        pltpu.make_async_copy(t2_hbm, ts2, sst).wait()
        pltpu.make_async_copy(t3_hbm, ts3, sst).wait()

    plsc.subcore_barrier()

    bufs = ((a1, a2, a3), (b1, b2, b3))
    gsems = (gs0, gs1)
    wsems = (ws0, ws1)

    def fire(t, p):
        cb = t * CH
        s = gsems[p]
        u1, u2, u3 = bufs[p]
        return (pltpu.async_copy(ts1.at[idx_d.at[pl.ds(cb, CH)]], u1, s),
                pltpu.async_copy(ts2.at[idx_s.at[pl.ds(cb, CH)]], u2, s),
                pltpu.async_copy(ts3.at[idx_e.at[pl.ds(cb, CH)]], u3, s))

    pend = {0: fire(0, 0)}
    wb = {}
    for t in range(NCHUNK):
        p = t % 2
        if t + 1 < NCHUNK:
            pn = (t + 1) % 2
            if pn in wb:
                wb.pop(pn).wait()
            pend[pn] = fire(t + 1, pn)
        for c in pend.pop(p):
            c.wait()
        u1, u2, u3 = bufs[p]

        def sum_body(r, carry):
            for v in range(D // L):
                sl = pl.ds(v * L, L)
                u1[r, sl] = u1[r, sl] + u2[r, sl] + u3[r, sl]
            return carry

        lax.fori_loop(0, CH, sum_body, 0)
        wb[p] = pltpu.async_copy(bufs[p][0],
                                 g_hbm.at[pl.ds(base + t * CH, CH), :],
                                 wsems[p])
    for c in wb.values():
        c.wait()


def _final_body(g_ref, cont_ref, fc1b_ref, fc2w_ref, ww_ref, fc2b_ref,
                wb_ref, o_ref):
    h = jnp.maximum(g_ref[...] + fc1b_ref[...], 0.0)
    o_ref[...] = (jnp.dot(h, fc2w_ref[...], preferred_element_type=jnp.float32)
                  + jnp.dot(cont_ref[...], ww_ref[...],
                            preferred_element_type=jnp.float32)
                  + fc2b_ref[...] + wb_ref[...])


BLK = 2048


def kernel(attr, wide_W, wide_b, dep_table, sid_table, eid_table,
           fc1_W, fc1_b, fc2_W, fc2_b):
    t1, t2, t3 = pl.pallas_call(
        _proj_body,
        out_shape=[jax.ShapeDtypeStruct((N_DEP, D), jnp.float32),
                   jax.ShapeDtypeStruct((N_SID, D), jnp.float32),
                   jax.ShapeDtypeStruct((N_SID, D), jnp.float32)],
    )(dep_table, sid_table, eid_table, fc1_W)

    attr_t = attr.T.reshape(-1)
    g = _gather_sum(t1, t2, t3, attr_t)

    cont = attr[:, 1:3]
    out = pl.pallas_call(
        _final_body,
        grid=(B // BLK,),
        in_specs=[
            pl.BlockSpec((BLK, D), lambda i: (i, 0)),
            pl.BlockSpec((BLK, 2), lambda i: (i, 0)),
            pl.BlockSpec((1, D), lambda i: (0, 0)),
            pl.BlockSpec((D, D), lambda i: (0, 0)),
            pl.BlockSpec((2, D), lambda i: (0, 0)),
            pl.BlockSpec((1, D), lambda i: (0, 0)),
            pl.BlockSpec((1, D), lambda i: (0, 0)),
        ],
        out_specs=pl.BlockSpec((BLK, D), lambda i: (i, 0)),
        out_shape=jax.ShapeDtypeStruct((B, D), jnp.float32),
    )(g, cont, fc1_b.reshape(1, D), fc2_W, wide_W, fc2_b.reshape(1, D),
      wide_b.reshape(1, D))
    return out


# speedup vs baseline: 4.9481x; 1.0827x over previous
import functools

import jax
import jax.numpy as jnp
from jax import lax
from jax.experimental import pallas as pl
from jax.experimental.pallas import tpu as pltpu
from jax.experimental.pallas import tpu_sc as plsc

B = 16384
D = 128
N_DEP = 144
N_SID = 1015
NC, NS, L = 2, 16, 16
NW = NC * NS
BPW = B // NW
CH = 128
NCHUNK = BPW // CH


def _proj_body(dep_ref, sid_ref, eid_ref, w_ref, t1_ref, t2_ref, t3_ref):
    t1_ref[...] = jnp.dot(dep_ref[...], w_ref[0:256, :],
                          preferred_element_type=jnp.float32)
    t2_ref[...] = jnp.dot(sid_ref[...], w_ref[256:512, :],
                          preferred_element_type=jnp.float32)
    t3_ref[...] = jnp.dot(eid_ref[...], w_ref[512:768, :],
                          preferred_element_type=jnp.float32)


_mesh = plsc.VectorSubcoreMesh(core_axis_name="c", subcore_axis_name="s",
                               num_cores=NC, num_subcores=NS)


@functools.partial(
    pl.kernel,
    out_type=jax.ShapeDtypeStruct((B, D), jnp.float32),
    mesh=_mesh,
    scratch_types=[
        pltpu.VMEM_SHARED((N_DEP, D), jnp.float32),
        pltpu.VMEM_SHARED((N_SID, D), jnp.float32),
        pltpu.VMEM_SHARED((N_SID, D), jnp.float32),
        pltpu.VMEM((BPW,), jnp.float32),
        pltpu.VMEM((BPW,), jnp.float32),
        pltpu.VMEM((BPW,), jnp.float32),
        pltpu.VMEM((BPW,), jnp.int32),
        pltpu.VMEM((BPW,), jnp.int32),
        pltpu.VMEM((BPW,), jnp.int32),
        pltpu.VMEM((CH, D), jnp.float32),
        pltpu.VMEM((CH, D), jnp.float32),
        pltpu.SemaphoreType.DMA,
        pltpu.SemaphoreType.DMA,
        pltpu.SemaphoreType.DMA,
        pltpu.SemaphoreType.DMA,
        pltpu.SemaphoreType.DMA,
    ],
)
def _gather_sum(t1_hbm, t2_hbm, t3_hbm, at_hbm, g_hbm,
                ts1, ts2, ts3,
                col_d, col_s, col_e, idx_d, idx_s, idx_e,
                u0, u1, sst, gs0, gs1, ws0, ws1):
    cid = lax.axis_index("c")
    sid = lax.axis_index("s")
    wid = sid * NC + cid
    base = wid * BPW

    @pl.when(sid == 0)
    def _stage():
        pltpu.async_copy(t1_hbm, ts1, sst)
        pltpu.async_copy(t2_hbm, ts2, sst)
        c3 = pltpu.async_copy(t3_hbm, ts3, sst)
        del c3

    pltpu.sync_copy(at_hbm.at[pl.ds(0 * B + base, BPW)], col_d)
    pltpu.sync_copy(at_hbm.at[pl.ds(3 * B + base, BPW)], col_s)
    pltpu.sync_copy(at_hbm.at[pl.ds(4 * B + base, BPW)], col_e)

    def build(i, carry):
        sl = pl.ds(i * L, L)
        idx_d[sl] = col_d[sl].astype(jnp.int32)
        idx_s[sl] = col_s[sl].astype(jnp.int32)
        idx_e[sl] = col_e[sl].astype(jnp.int32)
        return carry

    lax.fori_loop(0, BPW // L, build, 0)

    @pl.when(sid == 0)
    def _stage_wait():
        pltpu.make_async_copy(t1_hbm, ts1, sst).wait()
        pltpu.make_async_copy(t2_hbm, ts2, sst).wait()
        pltpu.make_async_copy(t3_hbm, ts3, sst).wait()

    plsc.subcore_barrier()

    bufs = (u0, u1)
    gsems = (gs0, gs1)
    wsems = (ws0, ws1)

    def fire1(t, p):
        cb = t * CH
        return pltpu.async_copy(ts1.at[idx_d.at[pl.ds(cb, CH)]], bufs[p],
                                gsems[p])

    def fire23(t, p):
        cb = t * CH
        return (pltpu.async_copy(ts2.at[idx_s.at[pl.ds(cb, CH)]], bufs[p],
                                 gsems[p], add=True),
                pltpu.async_copy(ts3.at[idx_e.at[pl.ds(cb, CH)]], bufs[p],
                                 gsems[p], add=True))

    pend1 = {0: fire1(0, 0)}
    wb = {}
    for t in range(NCHUNK):
        p = t % 2
        pend1.pop(p).wait()
        adds = fire23(t, p)
        if t + 1 < NCHUNK:
            pn = (t + 1) % 2
            if pn in wb:
                wb.pop(pn).wait()
            pend1[pn] = fire1(t + 1, pn)
        for c in adds:
            c.wait()
        wb[p] = pltpu.async_copy(bufs[p],
                                 g_hbm.at[pl.ds(base + t * CH, CH), :],
                                 wsems[p])
    for c in wb.values():
        c.wait()


def _final_body(g_ref, cont_ref, fc1b_ref, fc2w_ref, ww_ref, fc2b_ref,
                wb_ref, o_ref):
    h = jnp.maximum(g_ref[...] + fc1b_ref[...], 0.0)
    o_ref[...] = (jnp.dot(h, fc2w_ref[...], preferred_element_type=jnp.float32)
                  + jnp.dot(cont_ref[...], ww_ref[...],
                            preferred_element_type=jnp.float32)
                  + fc2b_ref[...] + wb_ref[...])


BLK = 2048


def kernel(attr, wide_W, wide_b, dep_table, sid_table, eid_table,
           fc1_W, fc1_b, fc2_W, fc2_b):
    t1, t2, t3 = pl.pallas_call(
        _proj_body,
        out_shape=[jax.ShapeDtypeStruct((N_DEP, D), jnp.float32),
                   jax.ShapeDtypeStruct((N_SID, D), jnp.float32),
                   jax.ShapeDtypeStruct((N_SID, D), jnp.float32)],
    )(dep_table, sid_table, eid_table, fc1_W)

    attr_t = attr.T.reshape(-1)
    g = _gather_sum(t1, t2, t3, attr_t)

    cont = attr[:, 1:3]
    out = pl.pallas_call(
        _final_body,
        grid=(B // BLK,),
        in_specs=[
            pl.BlockSpec((BLK, D), lambda i: (i, 0)),
            pl.BlockSpec((BLK, 2), lambda i: (i, 0)),
            pl.BlockSpec((1, D), lambda i: (0, 0)),
            pl.BlockSpec((D, D), lambda i: (0, 0)),
            pl.BlockSpec((2, D), lambda i: (0, 0)),
            pl.BlockSpec((1, D), lambda i: (0, 0)),
            pl.BlockSpec((1, D), lambda i: (0, 0)),
        ],
        out_specs=pl.BlockSpec((BLK, D), lambda i: (i, 0)),
        out_shape=jax.ShapeDtypeStruct((B, D), jnp.float32),
    )(g, cont, fc1_b.reshape(1, D), fc2_W, wide_W, fc2_b.reshape(1, D),
      wide_b.reshape(1, D))
    return out
